# baseline (speedup 1.0000x reference)
import functools

import jax
import jax.numpy as jnp
from jax import lax
from jax.experimental import pallas as pl
from jax.experimental.pallas import tpu as pltpu
from jax.experimental.pallas import tpu_sc as plsc

B, N, DIM, K = 16, 4096, 256, 1024
NC, NS, L = 2, 16, 16
_CHR = 128
_R, _C = 32, 128
_SGN = -2147483648


def _roll2(a, d, axis):
    return jnp.roll(a, d, axis=axis), jnp.roll(a, -d, axis=axis)


_G = 4
_NH = 8
_Q = K // 4


def _score_body(x_ref, w_ref, b_ref, nrm_ref, y_ref):
    xb = x_ref[0]
    wmat = w_ref[...]
    y128 = jnp.dot(xb.astype(jnp.bfloat16), wmat.astype(jnp.bfloat16),
                   preferred_element_type=jnp.float32)
    y_col = (y128[:, 0:1] + b_ref[0, 0]) / nrm_ref[0, 0]
    y_ref[0] = y_col.reshape(1, N)


def _score(x, wmat, b2, nrm):
    return pl.pallas_call(
        _score_body,
        grid=(B,),
        in_specs=[
            pl.BlockSpec((1, N, DIM), lambda i: (i, 0, 0)),
            pl.BlockSpec((DIM, 128), lambda i: (0, 0)),
            pl.BlockSpec(memory_space=pltpu.SMEM),
            pl.BlockSpec(memory_space=pltpu.SMEM),
        ],
        out_specs=pl.BlockSpec((1, 1, N), lambda i: (i, 0, 0)),
        out_shape=jax.ShapeDtypeStruct((B, 1, N), jnp.float32),
    )(x, wmat, b2, nrm)


def _sort_one(y):
    y = y.reshape(_R, _C)
    bits = lax.bitcast_convert_type(y, jnp.int32)
    ku = jnp.where(bits < 0, jnp.bitwise_not(bits ^ _SGN), bits)
    idx = (lax.broadcasted_iota(jnp.int32, (_R, _C), 0) * _C
           + lax.broadcasted_iota(jnp.int32, (_R, _C), 1))
    e = idx
    for p in range(1, 13):
        asc = (e & (1 << p)) == 0
        for q in range(p - 1, -1, -1):
            d = 1 << q
            upper = (e & d) != 0
            if d < _C:
                ka, kb = _roll2(ku, d, 1)
                ia, ib = _roll2(idx, d, 1)
            else:
                ka, kb = _roll2(ku, d // _C, 0)
                ia, ib = _roll2(idx, d // _C, 0)
            pk = jnp.where(upper, ka, kb)
            pi = jnp.where(upper, ia, ib)
            self_first = jnp.logical_or(
                ku > pk, jnp.logical_and(ku == pk, idx < pi))
            want_first = upper == jnp.logical_not(asc)
            take = want_first != self_first
            ku = jnp.where(take, pk, ku)
            idx = jnp.where(take, pi, idx)
    topi = idx[:K // _C, :].reshape(1, K)
    topk = ku[:K // _C, :]
    ybits = jnp.where(topk >= 0, topk, jnp.bitwise_not(topk ^ _SGN))
    ysel = lax.bitcast_convert_type(ybits, jnp.float32).reshape(1, K)
    return topi, 1.0 / (1.0 + jnp.exp(-ysel))


def _sort_body(y_ref, tidx_ref, ysel_ref):
    for g in range(_G):
        topi, ysel = _sort_one(y_ref[g])
        tidx_ref[g] = topi
        ysel_ref[g] = ysel


def _sort(y3):
    nb = y3.shape[0]
    return pl.pallas_call(
        _sort_body,
        grid=(nb // _G,),
        in_specs=[pl.BlockSpec((_G, 1, N), lambda i: (i, 0, 0))],
        out_specs=[
            pl.BlockSpec((_G, 1, K), lambda i: (i, 0, 0)),
            pl.BlockSpec((_G, 1, K), lambda i: (i, 0, 0)),
        ],
        out_shape=[
            jax.ShapeDtypeStruct((nb, 1, K), jnp.int32),
            jax.ShapeDtypeStruct((nb, 1, K), jnp.float32),
        ],
    )(y3)


def _sc_body(tidx_hbm, ysel_hbm, posf_hbm, xf_hbm,
             psel_hbm, xout_hbm,
             tq_v, ysel_v, gidx_v, pos_v, psel_v, xbuf_v, sem):
    cc_ = lax.axis_index("c")
    ss_ = lax.axis_index("s")
    wid = ss_ * NC + cc_
    b = wid // 4
    h = wid % 4
    hs = h * _Q
    lane = lax.broadcasted_iota(jnp.int32, (L,), 0)

    pltpu.sync_copy(tidx_hbm.at[pl.ds(b * K + hs, _Q)], tq_v)
    pltpu.sync_copy(ysel_hbm.at[pl.ds(b * K + hs, _Q)], ysel_v)

    def gidx_step(v, carry):
        off = pl.multiple_of(v * L, L)
        gidx_v[pl.ds(off, L)] = tq_v[pl.ds(off, L)] + b * N
        return carry
    lax.fori_loop(0, _Q // L, gidx_step, 0)

    pltpu.sync_copy(posf_hbm.at[b], pos_v)
    def pos_step(v, carry):
        off = pl.multiple_of(v * L, L)
        t16 = tq_v[pl.ds(off, L)]
        dst = lane + v * L
        for c3 in range(3):
            vals = plsc.load_gather(pos_v, [t16 * 3 + c3])
            plsc.store_scatter(psel_v, [dst * 3 + c3], vals)
        return carry
    lax.fori_loop(0, _Q // L, pos_step, 0)
    pltpu.sync_copy(psel_v, psel_hbm.at[pl.ds((b * K + hs) * 3, _Q * 3)])

    for ch in range(_Q // _CHR):
        pltpu.async_copy(
            xf_hbm.at[gidx_v.at[pl.ds(ch * _CHR, _CHR)]], xbuf_v, sem).wait()
        def scale_row(r, carry):
            sidx = jnp.full((L,), ch * _CHR, jnp.int32) + r
            srow = plsc.load_gather(ysel_v, [sidx])
            for j in range(DIM // L):
                xbuf_v[r, pl.ds(j * L, L)] = xbuf_v[r, pl.ds(j * L, L)] * srow
            return carry
        lax.fori_loop(0, _CHR, scale_row, 0)
        row0 = b * K + hs + ch * _CHR
        pltpu.sync_copy(xbuf_v, xout_hbm.at[pl.ds(row0, _CHR)])


@functools.lru_cache(maxsize=1)
def _sc_select():
    mesh = plsc.VectorSubcoreMesh(core_axis_name="c", subcore_axis_name="s",
                                  num_cores=NC, num_subcores=NS)
    return pl.kernel(
        _sc_body,
        out_type=[
            jax.ShapeDtypeStruct((_NH * K * 3,), jnp.float32),
            jax.ShapeDtypeStruct((_NH * K, DIM), jnp.float32),
        ],
        mesh=mesh,
        scratch_types=[
            pltpu.VMEM((_Q,), jnp.int32),
            pltpu.VMEM((_Q,), jnp.float32),
            pltpu.VMEM((_Q,), jnp.int32),
            pltpu.VMEM((3 * N,), jnp.float32),
            pltpu.VMEM((_Q * 3,), jnp.float32),
            pltpu.VMEM((_CHR, DIM), jnp.float32),
            pltpu.SemaphoreType.DMA,
        ],
        compiler_params=pltpu.CompilerParams(needs_layout_passes=False),
    )


def kernel(pos, x, W, b):
    nrm = jnp.linalg.norm(W).reshape(1, 1)
    b2 = b.reshape(1, 1)
    wmat = jnp.zeros((DIM, 128), jnp.float32).at[:, 0].set(W[0])
    y3 = _score(x, wmat, b2, nrm)
    posf = pos.reshape(B, 3 * N)
    halves = []
    for b0 in (0, _NH):
        tidx3, ysel3 = _sort(y3[b0:b0 + _NH])
        psel_f, xout_f = _sc_select()(
            tidx3.reshape(_NH * K), ysel3.reshape(_NH * K),
            posf[b0:b0 + _NH],
            x[b0:b0 + _NH].reshape(_NH * N, DIM))
        halves.append((tidx3.reshape(_NH, K),
                       psel_f.reshape(_NH, K, 3),
                       xout_f.reshape(_NH, K, DIM)))
    return (jnp.concatenate([halves[0][0], halves[1][0]], axis=0),
            jnp.concatenate([halves[0][1], halves[1][1]], axis=0),
            jnp.concatenate([halves[0][2], halves[1][2]], axis=0))


# speedup vs baseline: 1.4471x; 1.4471x over previous
import functools

import jax
import jax.numpy as jnp
from jax import lax
from jax.experimental import pallas as pl
from jax.experimental.pallas import tpu as pltpu
from jax.experimental.pallas import tpu_sc as plsc

B, N, DIM, K = 16, 4096, 256, 1024
NC, NS, L = 2, 16, 16
_CHR = 128
_R, _C = 32, 128
_SGN = -2147483648


def _roll2(a, d, axis):
    return jnp.roll(a, d, axis=axis), jnp.roll(a, -d, axis=axis)


_G = 4
_NH = B
_Q = K // 2


def _score_body(x_ref, w_ref, b_ref, nrm_ref, y_ref):
    xb = x_ref[0]
    wmat = w_ref[...]
    y128 = jnp.dot(xb.astype(jnp.bfloat16), wmat.astype(jnp.bfloat16),
                   preferred_element_type=jnp.float32)
    y_col = (y128[:, 0:1] + b_ref[0, 0]) / nrm_ref[0, 0]
    y_ref[0] = y_col.reshape(1, N)


def _score(x, wmat, b2, nrm):
    return pl.pallas_call(
        _score_body,
        grid=(B,),
        in_specs=[
            pl.BlockSpec((1, N, DIM), lambda i: (i, 0, 0)),
            pl.BlockSpec((DIM, 128), lambda i: (0, 0)),
            pl.BlockSpec(memory_space=pltpu.SMEM),
            pl.BlockSpec(memory_space=pltpu.SMEM),
        ],
        out_specs=pl.BlockSpec((1, 1, N), lambda i: (i, 0, 0)),
        out_shape=jax.ShapeDtypeStruct((B, 1, N), jnp.float32),
    )(x, wmat, b2, nrm)


def _sort_one(y):
    y = y.reshape(_R, _C)
    bits = lax.bitcast_convert_type(y, jnp.int32)
    ku = jnp.where(bits < 0, jnp.bitwise_not(bits ^ _SGN), bits)
    idx = (lax.broadcasted_iota(jnp.int32, (_R, _C), 0) * _C
           + lax.broadcasted_iota(jnp.int32, (_R, _C), 1))
    e = idx
    for p in range(1, 13):
        asc = (e & (1 << p)) == 0
        for q in range(p - 1, -1, -1):
            d = 1 << q
            upper = (e & d) != 0
            if d < _C:
                ka, kb = _roll2(ku, d, 1)
                ia, ib = _roll2(idx, d, 1)
            else:
                ka, kb = _roll2(ku, d // _C, 0)
                ia, ib = _roll2(idx, d // _C, 0)
            pk = jnp.where(upper, ka, kb)
            pi = jnp.where(upper, ia, ib)
            self_first = jnp.logical_or(
                ku > pk, jnp.logical_and(ku == pk, idx < pi))
            want_first = upper == jnp.logical_not(asc)
            take = want_first != self_first
            ku = jnp.where(take, pk, ku)
            idx = jnp.where(take, pi, idx)
    topi = idx[:K // _C, :].reshape(1, K)
    topk = ku[:K // _C, :]
    ybits = jnp.where(topk >= 0, topk, jnp.bitwise_not(topk ^ _SGN))
    ysel = lax.bitcast_convert_type(ybits, jnp.float32).reshape(1, K)
    return topi, 1.0 / (1.0 + jnp.exp(-ysel))


def _sort_body(y_ref, tidx_ref, ysel_ref):
    for g in range(_G):
        topi, ysel = _sort_one(y_ref[g])
        tidx_ref[g] = topi
        ysel_ref[g] = ysel


def _sort(y3):
    nb = y3.shape[0]
    return pl.pallas_call(
        _sort_body,
        grid=(nb // _G,),
        in_specs=[pl.BlockSpec((_G, 1, N), lambda i: (i, 0, 0))],
        out_specs=[
            pl.BlockSpec((_G, 1, K), lambda i: (i, 0, 0)),
            pl.BlockSpec((_G, 1, K), lambda i: (i, 0, 0)),
        ],
        out_shape=[
            jax.ShapeDtypeStruct((nb, 1, K), jnp.int32),
            jax.ShapeDtypeStruct((nb, 1, K), jnp.float32),
        ],
    )(y3)


def _sc_body(tidx_hbm, ysel_hbm, posf_hbm, xf_hbm,
             psel_hbm, xout_hbm,
             tq_v, ysel_v, gidx_v, pos_v, psel_v, xbuf_v, sem):
    cc_ = lax.axis_index("c")
    ss_ = lax.axis_index("s")
    wid = ss_ * NC + cc_
    b = wid // 2
    h = wid % 2
    hs = h * _Q
    lane = lax.broadcasted_iota(jnp.int32, (L,), 0)

    pltpu.sync_copy(tidx_hbm.at[pl.ds(b * K + hs, _Q)], tq_v)
    pltpu.sync_copy(ysel_hbm.at[pl.ds(b * K + hs, _Q)], ysel_v)

    def gidx_step(v, carry):
        off = pl.multiple_of(v * L, L)
        gidx_v[pl.ds(off, L)] = tq_v[pl.ds(off, L)] + b * N
        return carry
    lax.fori_loop(0, _Q // L, gidx_step, 0)

    pltpu.sync_copy(posf_hbm.at[b], pos_v)
    def pos_step(v, carry):
        off = pl.multiple_of(v * L, L)
        t16 = tq_v[pl.ds(off, L)]
        dst = lane + v * L
        for c3 in range(3):
            vals = plsc.load_gather(pos_v, [t16 * 3 + c3])
            plsc.store_scatter(psel_v, [dst * 3 + c3], vals)
        return carry
    lax.fori_loop(0, _Q // L, pos_step, 0)
    pltpu.sync_copy(psel_v, psel_hbm.at[pl.ds((b * K + hs) * 3, _Q * 3)])

    for ch in range(_Q // _CHR):
        pltpu.async_copy(
            xf_hbm.at[gidx_v.at[pl.ds(ch * _CHR, _CHR)]], xbuf_v, sem).wait()
        def scale_row(r, carry):
            sidx = jnp.full((L,), ch * _CHR, jnp.int32) + r
            srow = plsc.load_gather(ysel_v, [sidx])
            for j in range(DIM // L):
                xbuf_v[r, pl.ds(j * L, L)] = xbuf_v[r, pl.ds(j * L, L)] * srow
            return carry
        lax.fori_loop(0, _CHR, scale_row, 0)
        row0 = b * K + hs + ch * _CHR
        pltpu.sync_copy(xbuf_v, xout_hbm.at[pl.ds(row0, _CHR)])


@functools.lru_cache(maxsize=1)
def _sc_select():
    mesh = plsc.VectorSubcoreMesh(core_axis_name="c", subcore_axis_name="s",
                                  num_cores=NC, num_subcores=NS)
    return pl.kernel(
        _sc_body,
        out_type=[
            jax.ShapeDtypeStruct((_NH * K * 3,), jnp.float32),
            jax.ShapeDtypeStruct((_NH * K, DIM), jnp.float32),
        ],
        mesh=mesh,
        scratch_types=[
            pltpu.VMEM((_Q,), jnp.int32),
            pltpu.VMEM((_Q,), jnp.float32),
            pltpu.VMEM((_Q,), jnp.int32),
            pltpu.VMEM((3 * N,), jnp.float32),
            pltpu.VMEM((_Q * 3,), jnp.float32),
            pltpu.VMEM((_CHR, DIM), jnp.float32),
            pltpu.SemaphoreType.DMA,
        ],
        compiler_params=pltpu.CompilerParams(needs_layout_passes=False),
    )


def kernel(pos, x, W, b):
    nrm = jnp.linalg.norm(W).reshape(1, 1)
    b2 = b.reshape(1, 1)
    wmat = jnp.zeros((DIM, 128), jnp.float32).at[:, 0].set(W[0])
    y3 = _score(x, wmat, b2, nrm)
    posf = pos.reshape(B, 3 * N)
    tidx3, ysel3 = _sort(y3)
    psel_f, xout_f = _sc_select()(
        tidx3.reshape(B * K), ysel3.reshape(B * K),
        posf, x.reshape(B * N, DIM))
    return (tidx3.reshape(B, K),
            psel_f.reshape(B, K, 3),
            xout_f.reshape(B, K, DIM))
